# baseline (speedup 1.0000x reference)
import functools

import jax
import jax.numpy as jnp
from jax import lax
from jax.experimental import pallas as pl
from jax.experimental.pallas import tpu as pltpu
from jax.experimental.pallas import tpu_sc as plsc

N = 10000
NP = 10240
E = 320000
DP = 160
DH = DP // 2
LANES = 16
NC = 2
NS = 16
EDGES_PER_TILE = E // NS
CHUNK = 80
NCHUNK = EDGES_PER_TILE // CHUNK
ROWS_PER_SUB = NP // NS
TROWS_PER_SUB = N // NS


def _sc_aggregate(xb0, xb1, src, dst, attr, w, zeros):
    mesh = plsc.VectorSubcoreMesh(core_axis_name="c", subcore_axis_name="s")

    @functools.partial(
        pl.kernel,
        out_type=jax.ShapeDtypeStruct((NC, NP, DH), jnp.float32),
        mesh=mesh,
        compiler_params=pltpu.CompilerParams(use_tc_tiling_on_sc=False),
        scratch_types=[
            pltpu.VMEM_SHARED((NP, DH), jnp.float32),
            pltpu.VMEM_SHARED((N, DH), jnp.float32),
            pltpu.VMEM((CHUNK,), jnp.int32),
            pltpu.VMEM((CHUNK,), jnp.int32),
            pltpu.VMEM((CHUNK,), jnp.int32),
            pltpu.VMEM((CHUNK,), jnp.int32),
            pltpu.VMEM((CHUNK,), jnp.float32),
            pltpu.VMEM((CHUNK,), jnp.float32),
            pltpu.VMEM((CHUNK, DH), jnp.float32),
            pltpu.VMEM((CHUNK, DH), jnp.float32),
            pltpu.VMEM((CHUNK, DH), jnp.float32),
            pltpu.VMEM((CHUNK, DH), jnp.float32),
            pltpu.VMEM((DH,), jnp.float32),
            pltpu.SemaphoreType.DMA,
            pltpu.SemaphoreType.DMA,
            pltpu.SemaphoreType.DMA,
            pltpu.SemaphoreType.DMA,
            pltpu.SemaphoreType.DMA,
            pltpu.SemaphoreType.DMA,
        ],
    )
    def k(xb0_hbm, xb1_hbm, src_hbm, dst_hbm, attr_hbm, w_hbm, z_hbm,
          out_hbm, aggr_sh, table_sh, src0, src1, dst0, dst1, at0, at1,
          rows0, rows1, msg0, msg1, w_v,
          isem0, isem1, gsem0, gsem1, ssem0, ssem1):
        cid = lax.axis_index("c")
        sid = lax.axis_index("s")
        srcb = (src0, src1)
        dstb = (dst0, dst1)
        atb = (at0, at1)
        rowsb = (rows0, rows1)
        msgb = (msg0, msg1)
        isem = (isem0, isem1)
        gsem = (gsem0, gsem1)
        ssem = (ssem0, ssem1)

        t0 = sid * TROWS_PER_SUB

        @pl.when(cid == 0)
        def _():
            pltpu.sync_copy(xb0_hbm.at[pl.ds(t0, TROWS_PER_SUB)],
                            table_sh.at[pl.ds(t0, TROWS_PER_SUB)])

        @pl.when(cid == 1)
        def _():
            pltpu.sync_copy(xb1_hbm.at[pl.ds(t0, TROWS_PER_SUB)],
                            table_sh.at[pl.ds(t0, TROWS_PER_SUB)])

        pltpu.sync_copy(z_hbm, rows0)
        r0 = sid * ROWS_PER_SUB
        for rep in range(ROWS_PER_SUB // CHUNK):
            pltpu.async_copy(rows0, aggr_sh.at[pl.ds(r0 + rep * CHUNK, CHUNK)],
                             gsem0)
        for rep in range(ROWS_PER_SUB // CHUNK):
            pltpu.make_async_copy(rows0,
                                  aggr_sh.at[pl.ds(r0, CHUNK)], gsem0).wait()
        pltpu.sync_copy(w_hbm.at[cid], w_v)
        plsc.subcore_barrier()

        base = sid * EDGES_PER_TILE

        def issue_idx(kk, p):
            off = base + kk * CHUNK
            pltpu.async_copy(src_hbm.at[pl.ds(off, CHUNK)], srcb[p], isem[p])
            pltpu.async_copy(dst_hbm.at[pl.ds(off, CHUNK)], dstb[p], isem[p])
            pltpu.async_copy(attr_hbm.at[pl.ds(off, CHUNK)], atb[p], isem[p])

        def wait_idx(p):
            pltpu.make_async_copy(src_hbm.at[pl.ds(0, CHUNK)],
                                  srcb[p], isem[p]).wait()
            pltpu.make_async_copy(dst_hbm.at[pl.ds(0, CHUNK)],
                                  dstb[p], isem[p]).wait()
            pltpu.make_async_copy(attr_hbm.at[pl.ds(0, CHUNK)],
                                  atb[p], isem[p]).wait()

        def compute(p):
            rows_v, msg_v, attr_v = rowsb[p], msgb[p], atb[p]

            @plsc.parallel_loop(0, CHUNK // LANES)
            def _(g):
                a16 = attr_v[pl.ds(g * LANES, LANES)]
                for t in range(LANES):
                    a = a16[t]
                    i = g * LANES + t
                    for j in range(DH // LANES):
                        sl = pl.ds(j * LANES, LANES)
                        m = jnp.maximum(rows_v.at[i][sl] + a * w_v[sl], 0.0)
                        msg_v.at[i][sl] = m

        issue_idx(0, 0)
        wait_idx(0)
        pltpu.async_copy(table_sh.at[srcb[0]], rowsb[0], gsem[0])
        issue_idx(1, 1)

        @pl.loop(0, (NCHUNK + 1) // 2)
        def _(kkk):
            for par in range(2):
                kk = kkk * 2 + par
                p, p2 = par, 1 - par

                @pl.when(kk + 1 < NCHUNK)
                def _():
                    wait_idx(p2)
                    pltpu.async_copy(table_sh.at[srcb[p2]], rowsb[p2],
                                     gsem[p2])

                @pl.when(kk >= 2)
                def _():
                    pltpu.make_async_copy(
                        msgb[p], aggr_sh.at[dstb[p]], ssem[p]).wait()

                @pl.when(kk < NCHUNK)
                def _():
                    pltpu.make_async_copy(
                        table_sh.at[srcb[p]], rowsb[p], gsem[p]).wait()
                    compute(p)

                @pl.when(kk + 2 < NCHUNK)
                def _():
                    issue_idx(kk + 2, p)

                @pl.when(kk < NCHUNK)
                def _():
                    pltpu.async_copy(msgb[p], aggr_sh.at[dstb[p]],
                                     ssem[p], add=True)

        for q in (NCHUNK - 2, NCHUNK - 1):
            qp = q % 2
            pltpu.make_async_copy(msgb[qp], aggr_sh.at[dstb[qp]],
                                  ssem[qp]).wait()

        plsc.subcore_barrier()
        pltpu.sync_copy(aggr_sh.at[pl.ds(r0, ROWS_PER_SUB)],
                        out_hbm.at[cid, pl.ds(r0, ROWS_PER_SUB)])

    return k(xb0, xb1, src, dst, attr, w, zeros)


BLK = 1024


def _mlp_body(xp_ref, p0_ref, p1_ref, w1_ref, w1a_ref, w1b_ref,
              b1_ref, w2_ref, b2_ref, o_ref):
    z = lax.dot_general(xp_ref[...], w1_ref[...], (((1,), (0,)), ((), ())),
                        precision=lax.Precision.HIGHEST,
                        preferred_element_type=jnp.float32)
    z += lax.dot_general(p0_ref[...], w1a_ref[...], (((1,), (0,)), ((), ())),
                         precision=lax.Precision.HIGHEST,
                         preferred_element_type=jnp.float32)
    z += lax.dot_general(p1_ref[...], w1b_ref[...], (((1,), (0,)), ((), ())),
                         precision=lax.Precision.HIGHEST,
                         preferred_element_type=jnp.float32)
    z = jnp.maximum(z + b1_ref[...], 0.0)
    o = lax.dot_general(z, w2_ref[...], (((1,), (0,)), ((), ())),
                        precision=lax.Precision.HIGHEST,
                        preferred_element_type=jnp.float32)
    o_ref[...] = o + b2_ref[...]


def _tc_mlp(x_pad, p0, p1, w1p, w1a, w1b, b1, w2, b2):
    grid = (NP // BLK,)
    return pl.pallas_call(
        _mlp_body,
        grid=grid,
        in_specs=[
            pl.BlockSpec((BLK, DP), lambda i: (i, 0)),
            pl.BlockSpec((BLK, DH), lambda i: (i, 0)),
            pl.BlockSpec((BLK, DH), lambda i: (i, 0)),
            pl.BlockSpec((DP, 1024), lambda i: (0, 0)),
            pl.BlockSpec((DH, 1024), lambda i: (0, 0)),
            pl.BlockSpec((DH, 1024), lambda i: (0, 0)),
            pl.BlockSpec((1, 1024), lambda i: (0, 0)),
            pl.BlockSpec((1024, 64), lambda i: (0, 0)),
            pl.BlockSpec((1, 64), lambda i: (0, 0)),
        ],
        out_specs=pl.BlockSpec((BLK, 64), lambda i: (i, 0)),
        out_shape=jax.ShapeDtypeStruct((NP, 64), jnp.float32),
        compiler_params=pltpu.CompilerParams(
            dimension_semantics=("parallel",)),
    )(x_pad, p0, p1, w1p, w1a, w1b, b1, w2, b2)


def kernel(x, edge_index, edge_attr, rsig, W_edge, b_edge, W1, b1, W2, b2):
    x_in = jnp.concatenate([x, rsig], axis=-1)
    x_pad = jnp.pad(x_in, ((0, 0), (0, DP - x_in.shape[1])))
    w_pad = jnp.pad(W_edge[0], (0, DP - W_edge.shape[1]))
    b_pad = jnp.pad(b_edge, (0, DP - b_edge.shape[0]))
    xb_pad = x_pad + b_pad[None, :]
    xb0 = xb_pad[:, :DH]
    xb1 = xb_pad[:, DH:]
    wh = w_pad.reshape(NC, DH)
    src = edge_index[0]
    dst = edge_index[1]
    attr = edge_attr[:, 0]
    zeros = jnp.zeros((CHUNK, DH), jnp.float32)

    parts = _sc_aggregate(xb0, xb1, src, dst, attr, wh, zeros)

    x_pad_rows = jnp.pad(x_pad, ((0, NP - N), (0, 0)))
    w1p = jnp.pad(W1, ((0, DP - W1.shape[0]), (0, 0)))
    out = _tc_mlp(x_pad_rows, parts[0], parts[1], w1p,
                  w1p[:DH], w1p[DH:], b1.reshape(1, -1), W2,
                  b2.reshape(1, -1))
    return out[:N]


# speedup vs baseline: 1.9200x; 1.9200x over previous
import functools

import jax
import jax.numpy as jnp
from jax import lax
from jax.experimental import pallas as pl
from jax.experimental.pallas import tpu as pltpu
from jax.experimental.pallas import tpu_sc as plsc

N = 10000
NP = 10240
E = 320000
DP = 160
DH = DP // 2
LANES = 16
NC = 2
NS = 16
EDGES_PER_TILE = E // NS
CHUNK = 80
NCHUNK = EDGES_PER_TILE // CHUNK
ROWS_PER_SUB = NP // NS
TROWS_PER_SUB = N // NS


def _sc_aggregate(xb0, xb1, src, dst, attr, w, zeros):
    mesh = plsc.VectorSubcoreMesh(core_axis_name="c", subcore_axis_name="s")

    @functools.partial(
        pl.kernel,
        out_type=jax.ShapeDtypeStruct((NC, NP, DH), jnp.float32),
        mesh=mesh,
        compiler_params=pltpu.CompilerParams(use_tc_tiling_on_sc=False),
        scratch_types=[
            pltpu.VMEM_SHARED((NP, DH), jnp.float32),
            pltpu.VMEM_SHARED((N, DH), jnp.float32),
            pltpu.VMEM((CHUNK,), jnp.int32),
            pltpu.VMEM((CHUNK,), jnp.int32),
            pltpu.VMEM((CHUNK,), jnp.int32),
            pltpu.VMEM((CHUNK,), jnp.int32),
            pltpu.VMEM((CHUNK,), jnp.float32),
            pltpu.VMEM((CHUNK,), jnp.float32),
            pltpu.VMEM((CHUNK, DH), jnp.float32),
            pltpu.VMEM((CHUNK, DH), jnp.float32),
            pltpu.VMEM((CHUNK, DH), jnp.float32),
            pltpu.VMEM((CHUNK, DH), jnp.float32),
            pltpu.VMEM((DH,), jnp.float32),
            pltpu.SemaphoreType.DMA,
            pltpu.SemaphoreType.DMA,
            pltpu.SemaphoreType.DMA,
            pltpu.SemaphoreType.DMA,
            pltpu.SemaphoreType.DMA,
            pltpu.SemaphoreType.DMA,
        ],
    )
    def k(xb0_hbm, xb1_hbm, src_hbm, dst_hbm, attr_hbm, w_hbm, z_hbm,
          out_hbm, aggr_sh, table_sh, src0, src1, dst0, dst1, at0, at1,
          rows0, rows1, msg0, msg1, w_v,
          isem0, isem1, gsem0, gsem1, ssem0, ssem1):
        cid = lax.axis_index("c")
        sid = lax.axis_index("s")
        srcb = (src0, src1)
        dstb = (dst0, dst1)
        atb = (at0, at1)
        rowsb = (rows0, rows1)
        msgb = (msg0, msg1)
        isem = (isem0, isem1)
        gsem = (gsem0, gsem1)
        ssem = (ssem0, ssem1)

        t0 = sid * TROWS_PER_SUB

        @pl.when(cid == 0)
        def _():
            pltpu.sync_copy(xb0_hbm.at[pl.ds(t0, TROWS_PER_SUB)],
                            table_sh.at[pl.ds(t0, TROWS_PER_SUB)])

        @pl.when(cid == 1)
        def _():
            pltpu.sync_copy(xb1_hbm.at[pl.ds(t0, TROWS_PER_SUB)],
                            table_sh.at[pl.ds(t0, TROWS_PER_SUB)])

        pltpu.sync_copy(z_hbm, rows0)
        r0 = sid * ROWS_PER_SUB
        for rep in range(ROWS_PER_SUB // CHUNK):
            pltpu.async_copy(rows0, aggr_sh.at[pl.ds(r0 + rep * CHUNK, CHUNK)],
                             gsem0)
        for rep in range(ROWS_PER_SUB // CHUNK):
            pltpu.make_async_copy(rows0,
                                  aggr_sh.at[pl.ds(r0, CHUNK)], gsem0).wait()
        pltpu.sync_copy(w_hbm.at[cid], w_v)
        wregs = tuple(w_v[pl.ds(j * LANES, LANES)]
                      for j in range(DH // LANES))
        plsc.subcore_barrier()

        base = sid * EDGES_PER_TILE

        def issue_idx(kk, p):
            off = base + kk * CHUNK
            pltpu.async_copy(src_hbm.at[pl.ds(off, CHUNK)], srcb[p], isem[p])
            pltpu.async_copy(dst_hbm.at[pl.ds(off, CHUNK)], dstb[p], isem[p])
            pltpu.async_copy(attr_hbm.at[pl.ds(off, CHUNK)], atb[p], isem[p])

        def wait_idx(p):
            pltpu.make_async_copy(src_hbm.at[pl.ds(0, CHUNK)],
                                  srcb[p], isem[p]).wait()
            pltpu.make_async_copy(dst_hbm.at[pl.ds(0, CHUNK)],
                                  dstb[p], isem[p]).wait()
            pltpu.make_async_copy(attr_hbm.at[pl.ds(0, CHUNK)],
                                  atb[p], isem[p]).wait()

        def compute(p):
            rows_v, msg_v, attr_v = rowsb[p], msgb[p], atb[p]

            @plsc.parallel_loop(0, CHUNK // LANES)
            def _(g):
                a16 = attr_v[pl.ds(g * LANES, LANES)]
                for t in range(LANES):
                    a = a16[t]
                    i = g * LANES + t
                    for j in range(DH // LANES):
                        sl = pl.ds(j * LANES, LANES)
                        m = jnp.maximum(rows_v.at[i][sl] + a * wregs[j], 0.0)
                        msg_v.at[i][sl] = m

        issue_idx(0, 0)
        wait_idx(0)
        pltpu.async_copy(table_sh.at[srcb[0]], rowsb[0], gsem[0])
        issue_idx(1, 1)

        @pl.loop(0, (NCHUNK + 1) // 2)
        def _(kkk):
            for par in range(2):
                kk = kkk * 2 + par
                p, p2 = par, 1 - par

                @pl.when(kk + 1 < NCHUNK)
                def _():
                    wait_idx(p2)
                    pltpu.async_copy(table_sh.at[srcb[p2]], rowsb[p2],
                                     gsem[p2])

                @pl.when(kk >= 2)
                def _():
                    pltpu.make_async_copy(
                        msgb[p], aggr_sh.at[dstb[p]], ssem[p]).wait()

                @pl.when(kk < NCHUNK)
                def _():
                    pltpu.make_async_copy(
                        table_sh.at[srcb[p]], rowsb[p], gsem[p]).wait()
                    compute(p)

                @pl.when(kk + 2 < NCHUNK)
                def _():
                    issue_idx(kk + 2, p)

                @pl.when(kk < NCHUNK)
                def _():
                    pltpu.async_copy(msgb[p], aggr_sh.at[dstb[p]],
                                     ssem[p], add=True)

        for q in (NCHUNK - 2, NCHUNK - 1):
            qp = q % 2
            pltpu.make_async_copy(msgb[qp], aggr_sh.at[dstb[qp]],
                                  ssem[qp]).wait()

        plsc.subcore_barrier()
        pltpu.sync_copy(aggr_sh.at[pl.ds(r0, ROWS_PER_SUB)],
                        out_hbm.at[cid, pl.ds(r0, ROWS_PER_SUB)])

    return k(xb0, xb1, src, dst, attr, w, zeros)


BLK = 1024


def _mlp_body(xp_ref, p0_ref, p1_ref, w1_ref, w1a_ref, w1b_ref,
              b1_ref, w2_ref, b2_ref, o_ref):
    z = lax.dot_general(xp_ref[...], w1_ref[...], (((1,), (0,)), ((), ())),
                        precision=lax.Precision.HIGHEST,
                        preferred_element_type=jnp.float32)
    z += lax.dot_general(p0_ref[...], w1a_ref[...], (((1,), (0,)), ((), ())),
                         precision=lax.Precision.HIGHEST,
                         preferred_element_type=jnp.float32)
    z += lax.dot_general(p1_ref[...], w1b_ref[...], (((1,), (0,)), ((), ())),
                         precision=lax.Precision.HIGHEST,
                         preferred_element_type=jnp.float32)
    z = jnp.maximum(z + b1_ref[...], 0.0)
    o = lax.dot_general(z, w2_ref[...], (((1,), (0,)), ((), ())),
                        precision=lax.Precision.HIGHEST,
                        preferred_element_type=jnp.float32)
    o_ref[...] = o + b2_ref[...]


def _tc_mlp(x_pad, p0, p1, w1p, w1a, w1b, b1, w2, b2):
    grid = (NP // BLK,)
    return pl.pallas_call(
        _mlp_body,
        grid=grid,
        in_specs=[
            pl.BlockSpec((BLK, DP), lambda i: (i, 0)),
            pl.BlockSpec((BLK, DH), lambda i: (i, 0)),
            pl.BlockSpec((BLK, DH), lambda i: (i, 0)),
            pl.BlockSpec((DP, 1024), lambda i: (0, 0)),
            pl.BlockSpec((DH, 1024), lambda i: (0, 0)),
            pl.BlockSpec((DH, 1024), lambda i: (0, 0)),
            pl.BlockSpec((1, 1024), lambda i: (0, 0)),
            pl.BlockSpec((1024, 64), lambda i: (0, 0)),
            pl.BlockSpec((1, 64), lambda i: (0, 0)),
        ],
        out_specs=pl.BlockSpec((BLK, 64), lambda i: (i, 0)),
        out_shape=jax.ShapeDtypeStruct((NP, 64), jnp.float32),
        compiler_params=pltpu.CompilerParams(
            dimension_semantics=("parallel",)),
    )(x_pad, p0, p1, w1p, w1a, w1b, b1, w2, b2)


def kernel(x, edge_index, edge_attr, rsig, W_edge, b_edge, W1, b1, W2, b2):
    x_in = jnp.concatenate([x, rsig], axis=-1)
    x_pad = jnp.pad(x_in, ((0, 0), (0, DP - x_in.shape[1])))
    w_pad = jnp.pad(W_edge[0], (0, DP - W_edge.shape[1]))
    b_pad = jnp.pad(b_edge, (0, DP - b_edge.shape[0]))
    xb_pad = x_pad + b_pad[None, :]
    xb0 = xb_pad[:, :DH]
    xb1 = xb_pad[:, DH:]
    wh = w_pad.reshape(NC, DH)
    src = edge_index[0]
    dst = edge_index[1]
    attr = edge_attr[:, 0]
    zeros = jnp.zeros((CHUNK, DH), jnp.float32)

    parts = _sc_aggregate(xb0, xb1, src, dst, attr, wh, zeros)

    x_pad_rows = jnp.pad(x_pad, ((0, NP - N), (0, 0)))
    w1p = jnp.pad(W1, ((0, DP - W1.shape[0]), (0, 0)))
    out = _tc_mlp(x_pad_rows, parts[0], parts[1], w1p,
                  w1p[:DH], w1p[DH:], b1.reshape(1, -1), W2,
                  b2.reshape(1, -1))
    return out[:N]


# speedup vs baseline: 2.1487x; 1.1191x over previous
import functools

import jax
import jax.numpy as jnp
from jax import lax
from jax.experimental import pallas as pl
from jax.experimental.pallas import tpu as pltpu
from jax.experimental.pallas import tpu_sc as plsc

N = 10000
NP = 10240
E = 320000
DP = 160
DH = DP // 2
LANES = 16
NC = 2
NS = 16
EDGES_PER_TILE = E // NS
CHUNK = 80
NCHUNK = EDGES_PER_TILE // CHUNK
ROWS_PER_SUB = NP // NS
TROWS_PER_SUB = N // NS


def _sc_aggregate(xcat, edata, w, zeros):
    mesh = plsc.VectorSubcoreMesh(core_axis_name="c", subcore_axis_name="s")

    @functools.partial(
        pl.kernel,
        out_type=jax.ShapeDtypeStruct((NC, NP, DH), jnp.float32),
        mesh=mesh,
        compiler_params=pltpu.CompilerParams(
            use_tc_tiling_on_sc=False, needs_layout_passes=False),
        scratch_types=[
            pltpu.VMEM_SHARED((NP, DH), jnp.float32),
            pltpu.VMEM_SHARED((N, DH), jnp.float32),
            pltpu.VMEM((3, CHUNK), jnp.int32),
            pltpu.VMEM((3, CHUNK), jnp.int32),
            pltpu.VMEM((CHUNK, DH), jnp.float32),
            pltpu.VMEM((CHUNK, DH), jnp.float32),
            pltpu.VMEM((CHUNK, DH), jnp.float32),
            pltpu.VMEM((CHUNK, DH), jnp.float32),
            pltpu.VMEM((DH,), jnp.float32),
            pltpu.SemaphoreType.DMA,
            pltpu.SemaphoreType.DMA,
            pltpu.SemaphoreType.DMA,
            pltpu.SemaphoreType.DMA,
            pltpu.SemaphoreType.DMA,
            pltpu.SemaphoreType.DMA,
        ],
    )
    def k(xcat_hbm, ed_hbm, w_hbm, z_hbm,
          out_hbm, aggr_sh, table_sh, ed0, ed1,
          rows0, rows1, msg0, msg1, w_v,
          isem0, isem1, gsem0, gsem1, ssem0, ssem1):
        cid = lax.axis_index("c")
        sid = lax.axis_index("s")
        edb = (ed0, ed1)
        rowsb = (rows0, rows1)
        msgb = (msg0, msg1)
        isem = (isem0, isem1)
        gsem = (gsem0, gsem1)
        ssem = (ssem0, ssem1)

        t0 = sid * TROWS_PER_SUB
        c0 = cid * DH
        pltpu.sync_copy(xcat_hbm.at[pl.ds(t0, TROWS_PER_SUB), pl.ds(c0, DH)],
                        table_sh.at[pl.ds(t0, TROWS_PER_SUB)])

        pltpu.sync_copy(z_hbm, rows0)
        r0 = sid * ROWS_PER_SUB
        for rep in range(ROWS_PER_SUB // CHUNK):
            pltpu.async_copy(rows0, aggr_sh.at[pl.ds(r0 + rep * CHUNK, CHUNK)],
                             gsem0)
        for rep in range(ROWS_PER_SUB // CHUNK):
            pltpu.make_async_copy(rows0,
                                  aggr_sh.at[pl.ds(r0, CHUNK)], gsem0).wait()
        pltpu.sync_copy(w_hbm.at[cid], w_v)
        wregs = tuple(w_v[pl.ds(j * LANES, LANES)]
                      for j in range(DH // LANES))
        plsc.subcore_barrier()

        cbase = sid * NCHUNK

        def issue_idx(kk, p):
            pltpu.async_copy(ed_hbm.at[cbase + kk], edb[p], isem[p])

        def wait_idx(p):
            pltpu.make_async_copy(ed_hbm.at[0], edb[p], isem[p]).wait()

        def compute(p):
            rows_v, msg_v, ed_v = rowsb[p], msgb[p], edb[p]

            @plsc.parallel_loop(0, CHUNK // LANES, unroll=2)
            def _(g):
                a16 = plsc.bitcast(ed_v[2, pl.ds(g * LANES, LANES)],
                                   jnp.float32)
                for t in range(LANES):
                    a = a16[t]
                    i = g * LANES + t
                    for j in range(DH // LANES):
                        sl = pl.ds(j * LANES, LANES)
                        m = jnp.maximum(rows_v.at[i][sl] + a * wregs[j], 0.0)
                        msg_v.at[i][sl] = m

        issue_idx(0, 0)
        wait_idx(0)
        pltpu.async_copy(table_sh.at[edb[0].at[0]], rowsb[0], gsem[0])
        issue_idx(1, 1)

        @pl.loop(0, (NCHUNK + 1) // 2)
        def _(kkk):
            for par in range(2):
                kk = kkk * 2 + par
                p, p2 = par, 1 - par

                @pl.when(kk + 1 < NCHUNK)
                def _():
                    wait_idx(p2)
                    pltpu.async_copy(table_sh.at[edb[p2].at[0]],
                                     rowsb[p2], gsem[p2])

                @pl.when(kk >= 2)
                def _():
                    pltpu.make_async_copy(
                        msgb[p], aggr_sh.at[edb[p].at[1]],
                        ssem[p]).wait()

                @pl.when(kk < NCHUNK)
                def _():
                    pltpu.make_async_copy(
                        table_sh.at[edb[p].at[0]], rowsb[p], gsem[p]).wait()
                    compute(p)

                @pl.when(kk + 2 < NCHUNK)
                def _():
                    issue_idx(kk + 2, p)

                @pl.when(kk < NCHUNK)
                def _():
                    pltpu.async_copy(msgb[p], aggr_sh.at[edb[p].at[1]],
                                     ssem[p], add=True)

        for q in (NCHUNK - 2, NCHUNK - 1):
            qp = q % 2
            pltpu.make_async_copy(msgb[qp], aggr_sh.at[edb[qp].at[1]],
                                  ssem[qp]).wait()

        plsc.subcore_barrier()
        pltpu.sync_copy(aggr_sh.at[pl.ds(r0, ROWS_PER_SUB)],
                        out_hbm.at[cid, pl.ds(r0, ROWS_PER_SUB)])

    return k(xcat, edata, w, zeros)


BLK = 1000


def _mlp_body(xp_ref, p0_ref, p1_ref, w1_ref, w1a_ref, w1b_ref,
              b1_ref, w2_ref, b2_ref, o_ref):
    z = lax.dot_general(xp_ref[...], w1_ref[...], (((1,), (0,)), ((), ())),
                        precision=lax.Precision.HIGHEST,
                        preferred_element_type=jnp.float32)
    z += lax.dot_general(p0_ref[...], w1a_ref[...], (((1,), (0,)), ((), ())),
                         precision=lax.Precision.HIGHEST,
                         preferred_element_type=jnp.float32)
    z += lax.dot_general(p1_ref[...], w1b_ref[...], (((1,), (0,)), ((), ())),
                         precision=lax.Precision.HIGHEST,
                         preferred_element_type=jnp.float32)
    z = jnp.maximum(z + b1_ref[...], 0.0)
    o = lax.dot_general(z, w2_ref[...], (((1,), (0,)), ((), ())),
                        precision=lax.Precision.HIGHEST,
                        preferred_element_type=jnp.float32)
    o_ref[...] = o + b2_ref[...]


def _tc_mlp(x_pad, p0, p1, w1p, w1a, w1b, b1, w2, b2):
    grid = (N // BLK,)
    return pl.pallas_call(
        _mlp_body,
        grid=grid,
        in_specs=[
            pl.BlockSpec((BLK, DP), lambda i: (i, 0)),
            pl.BlockSpec((BLK, DH), lambda i: (i, 0)),
            pl.BlockSpec((BLK, DH), lambda i: (i, 0)),
            pl.BlockSpec((DP, 1024), lambda i: (0, 0)),
            pl.BlockSpec((DH, 1024), lambda i: (0, 0)),
            pl.BlockSpec((DH, 1024), lambda i: (0, 0)),
            pl.BlockSpec((1, 1024), lambda i: (0, 0)),
            pl.BlockSpec((1024, 64), lambda i: (0, 0)),
            pl.BlockSpec((1, 64), lambda i: (0, 0)),
        ],
        out_specs=pl.BlockSpec((BLK, 64), lambda i: (i, 0)),
        out_shape=jax.ShapeDtypeStruct((N, 64), jnp.float32),
        compiler_params=pltpu.CompilerParams(
            dimension_semantics=("parallel",)),
    )(x_pad, p0, p1, w1p, w1a, w1b, b1, w2, b2)


def kernel(x, edge_index, edge_attr, rsig, W_edge, b_edge, W1, b1, W2, b2):
    x_in = jnp.concatenate([x, rsig], axis=-1)
    w_pad = jnp.pad(W_edge[0], (0, DP - W_edge.shape[1]))
    b_pad = jnp.pad(b_edge, (0, DP - b_edge.shape[0]))
    xcat = jnp.pad(x_in + b_edge[None, :],
                   ((0, 0), (0, DP - x_in.shape[1])))
    wh = w_pad.reshape(NC, DH)
    sc_ = edge_index[0].reshape(E // CHUNK, 1, CHUNK)
    dc_ = edge_index[1].reshape(E // CHUNK, 1, CHUNK)
    ac_ = lax.bitcast_convert_type(edge_attr[:, 0],
                                   jnp.int32).reshape(E // CHUNK, 1, CHUNK)
    edata = jnp.concatenate([sc_, dc_, ac_], axis=1)
    zeros = jnp.zeros((CHUNK, DH), jnp.float32)

    parts = _sc_aggregate(xcat, edata, wh, zeros)

    w1p = jnp.pad(W1, ((0, DP - W1.shape[0]), (0, 0)))
    b1_eff = b1 - b_pad @ w1p
    out = _tc_mlp(xcat, parts[0], parts[1], w1p,
                  w1p[:DH], w1p[DH:], b1_eff.reshape(1, -1), W2,
                  b2.reshape(1, -1))
    return out
